# initial kernel scaffold (unmeasured)
import jax
import jax.numpy as jnp
from jax import lax
from jax.experimental import pallas as pl
from jax.experimental.pallas import tpu as pltpu

N_DEV = 4
SCALE = 0.08838834764831843
NEG = -1e9


def kernel(x, Wq, K_ext, V_ext, Wo):
    B, Sq, D = x.shape
    _, Skv, H, Dh = K_ext.shape
    assert H * Dh == D

    x2 = x.reshape(Sq, D)
    k2 = K_ext.reshape(Skv, D)
    v2 = V_ext.reshape(Skv, D)

    def body(x_ref, wq_ref, k_hbm, v_hbm, wo_ref, out_ref,
             kbf, vbf, stage, qc, ac, st,
             copy_sem, q_send, a_send, s_send, q_recv, a_recv, s_recv):
        my = lax.axis_index("i")
        right = lax.rem(my + 1, N_DEV)

        cp = pltpu.make_async_copy(k_hbm, stage, copy_sem)
        cp.start()
        cp.wait()
        kbf[...] = stage[...].astype(jnp.bfloat16)
        cp = pltpu.make_async_copy(v_hbm, stage, copy_sem)
        cp.start()
        cp.wait()
        vbf[...] = stage[...].astype(jnp.bfloat16)

        qi = lax.broadcasted_iota(jnp.int32, (Sq, Skv), 0)
        ki = lax.broadcasted_iota(jnp.int32, (Sq, Skv), 1)
        addmask = jnp.where((qi // 64) % 4 == (ki // 64) % 4,
                            jnp.float32(0.0), jnp.float32(NEG))

        q = jnp.dot(x_ref[...], wq_ref[...], preferred_element_type=jnp.float32)
        qc[0] = (q * SCALE).astype(jnp.bfloat16)
        ac[0] = jnp.zeros((Sq, D), jnp.float32)
        st[0, :, 0:H] = jnp.full((Sq, H), -1e30, jnp.float32)
        st[0, :, H:2 * H] = jnp.zeros((Sq, H), jnp.float32)

        for s in range(N_DEV):
            for h in range(H):
                hs = slice(h * Dh, (h + 1) * Dh)
                scores = lax.dot_general(
                    qc[s, :, hs], kbf[:, hs],
                    (((1,), (1,)), ((), ())),
                    preferred_element_type=jnp.float32)
                scores = scores + addmask
                m_old = st[s, :, h:h + 1]
                l_old = st[s, :, H + h:H + h + 1]
                m_new = jnp.maximum(m_old, jnp.max(scores, axis=1, keepdims=True))
                alpha = jnp.exp(m_old - m_new)
                p = jnp.exp(scores - m_new)
                l_new = l_old * alpha + jnp.sum(p, axis=1, keepdims=True)
                pv = lax.dot_general(
                    p.astype(jnp.bfloat16), vbf[:, hs],
                    (((1,), (0,)), ((), ())),
                    preferred_element_type=jnp.float32)
                ac[s, :, hs] = ac[s, :, hs] * alpha + pv
                st[s, :, h:h + 1] = m_new
                st[s, :, H + h:H + h + 1] = l_new

            rq = pltpu.make_async_remote_copy(
                src_ref=qc.at[s], dst_ref=qc.at[s + 1],
                send_sem=q_send.at[s], recv_sem=q_recv.at[s],
                device_id=(right,), device_id_type=pl.DeviceIdType.MESH)
            ra = pltpu.make_async_remote_copy(
                src_ref=ac.at[s], dst_ref=ac.at[s + 1],
                send_sem=a_send.at[s], recv_sem=a_recv.at[s],
                device_id=(right,), device_id_type=pl.DeviceIdType.MESH)
            rs = pltpu.make_async_remote_copy(
                src_ref=st.at[s], dst_ref=st.at[s + 1],
                send_sem=s_send.at[s], recv_sem=s_recv.at[s],
                device_id=(right,), device_id_type=pl.DeviceIdType.MESH)
            rq.start()
            ra.start()
            rs.start()
            rq.wait()
            ra.wait()
            rs.wait()

        for h in range(H):
            hs = slice(h * Dh, (h + 1) * Dh)
            ac[N_DEV, :, hs] = ac[N_DEV, :, hs] / st[N_DEV, :, H + h:H + h + 1]
        out_ref[...] = jnp.dot(ac[N_DEV], wo_ref[...],
                               preferred_element_type=jnp.float32)

    out = pl.pallas_call(
        body,
        out_shape=jax.ShapeDtypeStruct((Sq, D), jnp.float32),
        in_specs=[
            pl.BlockSpec(memory_space=pltpu.VMEM),
            pl.BlockSpec(memory_space=pltpu.VMEM),
            pl.BlockSpec(memory_space=pltpu.ANY),
            pl.BlockSpec(memory_space=pltpu.ANY),
            pl.BlockSpec(memory_space=pltpu.VMEM),
        ],
        out_specs=pl.BlockSpec(memory_space=pltpu.VMEM),
        scratch_shapes=[
            pltpu.VMEM((Skv, D), jnp.bfloat16),
            pltpu.VMEM((Skv, D), jnp.bfloat16),
            pltpu.VMEM((Skv, D), jnp.float32),
            pltpu.VMEM((N_DEV + 1, Sq, D), jnp.bfloat16),
            pltpu.VMEM((N_DEV + 1, Sq, D), jnp.float32),
            pltpu.VMEM((N_DEV + 1, Sq, 2 * H), jnp.float32),
            pltpu.SemaphoreType.DMA,
            pltpu.SemaphoreType.DMA((N_DEV,)),
            pltpu.SemaphoreType.DMA((N_DEV,)),
            pltpu.SemaphoreType.DMA((N_DEV,)),
            pltpu.SemaphoreType.DMA((N_DEV,)),
            pltpu.SemaphoreType.DMA((N_DEV,)),
            pltpu.SemaphoreType.DMA((N_DEV,)),
        ],
        compiler_params=pltpu.CompilerParams(collective_id=0),
    )(x2, Wq, k2, v2, Wo)
    return out.reshape(B, Sq, D)


# baseline (device time: 178460 ns/iter reference)
import jax
import jax.numpy as jnp
from jax import lax
from jax.experimental import pallas as pl
from jax.experimental.pallas import tpu as pltpu

N_DEV = 4
SCALE = 0.08838834764831843
BLK = 64
NGRP = 4


def kernel(x, Wq, K_ext, V_ext, Wo):
    B, Sq, D = x.shape
    _, Skv, H, Dh = K_ext.shape
    assert H * Dh == D
    G = Skv // NGRP
    NQB = Sq // BLK

    x2 = x.reshape(Sq, D)
    k2 = K_ext.reshape(Skv, D)
    v2 = V_ext.reshape(Skv, D)

    C = 512

    def body(x_ref, wq_ref, k_hbm, v_hbm, wo_ref, out_ref,
             kbf, vbf, stage, qc, ac, st,
             copy_sem, q_send, a_send, s_send, q_recv, a_recv, s_recv):
        my = lax.axis_index("i")
        right = lax.rem(my + 1, N_DEV)

        for src, dst in ((k_hbm, kbf), (v_hbm, vbf)):
            for c in range(Skv // C):
                cp = pltpu.make_async_copy(
                    src.at[pl.ds(c * C, C)], stage, copy_sem)
                cp.start()
                cp.wait()
                for b in range(C // BLK):
                    t = c * (C // BLK) + b
                    row = (t % NGRP) * G + (t // NGRP) * BLK
                    dst[pl.ds(row, BLK), :] = (
                        stage[pl.ds(b * BLK, BLK), :].astype(jnp.bfloat16))

        q = jnp.dot(x_ref[...], wq_ref[...], preferred_element_type=jnp.float32)
        qc[0] = (q * SCALE).astype(jnp.bfloat16)
        ac[0] = jnp.zeros((Sq, D), jnp.bfloat16)
        st[0, :, 0:H] = jnp.full((Sq, H), -1e30, jnp.float32)
        st[0, :, H:2 * H] = jnp.zeros((Sq, H), jnp.float32)

        for s in range(N_DEV):
            for j in range(NQB):
                qs = pl.ds(j * BLK, BLK)
                gs = pl.ds(j * G, G)
                for h in range(H):
                    hs = slice(h * Dh, (h + 1) * Dh)
                    scores = lax.dot_general(
                        qc[s, qs, hs], kbf[gs, hs],
                        (((1,), (1,)), ((), ())),
                        preferred_element_type=jnp.float32)
                    m_old = st[s, qs, h:h + 1]
                    l_old = st[s, qs, H + h:H + h + 1]
                    m_new = jnp.maximum(
                        m_old, jnp.max(scores, axis=1, keepdims=True))
                    alpha = jnp.exp(m_old - m_new)
                    p = jnp.exp(scores - m_new)
                    l_new = l_old * alpha + jnp.sum(p, axis=1, keepdims=True)
                    pv = lax.dot_general(
                        p.astype(jnp.bfloat16), vbf[gs, hs],
                        (((1,), (0,)), ((), ())),
                        preferred_element_type=jnp.float32)
                    ac[s, qs, hs] = (ac[s, qs, hs].astype(jnp.float32) * alpha
                                     + pv).astype(jnp.bfloat16)
                    st[s, qs, h:h + 1] = m_new
                    st[s, qs, H + h:H + h + 1] = l_new

            rq = pltpu.make_async_remote_copy(
                src_ref=qc.at[s], dst_ref=qc.at[s + 1],
                send_sem=q_send.at[s], recv_sem=q_recv.at[s],
                device_id=(right,), device_id_type=pl.DeviceIdType.MESH)
            ra = pltpu.make_async_remote_copy(
                src_ref=ac.at[s], dst_ref=ac.at[s + 1],
                send_sem=a_send.at[s], recv_sem=a_recv.at[s],
                device_id=(right,), device_id_type=pl.DeviceIdType.MESH)
            rs = pltpu.make_async_remote_copy(
                src_ref=st.at[s], dst_ref=st.at[s + 1],
                send_sem=s_send.at[s], recv_sem=s_recv.at[s],
                device_id=(right,), device_id_type=pl.DeviceIdType.MESH)
            rq.start()
            ra.start()
            rs.start()
            rq.wait()
            ra.wait()
            rs.wait()

        for h in range(H):
            hs = slice(h * Dh, (h + 1) * Dh)
            ctx = (ac[N_DEV, :, hs].astype(jnp.float32)
                   / st[N_DEV, :, H + h:H + h + 1])
            ac[N_DEV, :, hs] = ctx.astype(jnp.bfloat16)
        out_ref[...] = jnp.dot(ac[N_DEV], wo_ref[...].astype(jnp.bfloat16),
                               preferred_element_type=jnp.float32)

    out = pl.pallas_call(
        body,
        out_shape=jax.ShapeDtypeStruct((Sq, D), jnp.float32),
        in_specs=[
            pl.BlockSpec(memory_space=pltpu.VMEM),
            pl.BlockSpec(memory_space=pltpu.VMEM),
            pl.BlockSpec(memory_space=pl.ANY),
            pl.BlockSpec(memory_space=pl.ANY),
            pl.BlockSpec(memory_space=pltpu.VMEM),
        ],
        out_specs=pl.BlockSpec(memory_space=pltpu.VMEM),
        scratch_shapes=[
            pltpu.VMEM((Skv, D), jnp.bfloat16),
            pltpu.VMEM((Skv, D), jnp.bfloat16),
            pltpu.VMEM((C, D), jnp.float32),
            pltpu.VMEM((N_DEV + 1, Sq, D), jnp.bfloat16),
            pltpu.VMEM((N_DEV + 1, Sq, D), jnp.bfloat16),
            pltpu.VMEM((N_DEV + 1, Sq, 2 * H), jnp.float32),
            pltpu.SemaphoreType.DMA,
            pltpu.SemaphoreType.DMA((N_DEV,)),
            pltpu.SemaphoreType.DMA((N_DEV,)),
            pltpu.SemaphoreType.DMA((N_DEV,)),
            pltpu.SemaphoreType.DMA((N_DEV,)),
            pltpu.SemaphoreType.DMA((N_DEV,)),
            pltpu.SemaphoreType.DMA((N_DEV,)),
        ],
    )(x2, Wq, k2, v2, Wo)
    return out.reshape(B, Sq, D)


# device time: 153826 ns/iter; 1.1601x vs baseline; 1.1601x over previous
import jax
import jax.numpy as jnp
from jax import lax
from jax.experimental import pallas as pl
from jax.experimental.pallas import tpu as pltpu

N_DEV = 4
SCALE = 0.08838834764831843
BLK = 64
NGRP = 4


def kernel(x, Wq, K_ext, V_ext, Wo):
    B, Sq, D = x.shape
    _, Skv, H, Dh = K_ext.shape
    assert H * Dh == D
    G = Skv // NGRP
    NQB = Sq // BLK

    x2 = x.reshape(Sq, D).astype(jnp.bfloat16)
    wq2 = Wq.astype(jnp.bfloat16)
    wo2 = Wo.astype(jnp.bfloat16)
    kp = (K_ext.reshape(Skv // BLK // NGRP, NGRP, BLK, D)
          .transpose(1, 0, 2, 3).reshape(Skv, D).astype(jnp.bfloat16))
    vp = (V_ext.reshape(Skv // BLK // NGRP, NGRP, BLK, D)
          .transpose(1, 0, 2, 3).reshape(Skv, D).astype(jnp.bfloat16))

    def body(x_ref, wq_ref, kbf, vbf, wo_ref, out_ref,
             qc, ac, st,
             q_send, a_send, s_send, q_recv, a_recv, s_recv):
        my = lax.axis_index("i")
        right = lax.rem(my + 1, N_DEV)

        q = jnp.dot(x_ref[...], wq_ref[...], preferred_element_type=jnp.float32)
        qc[0] = (q * SCALE).astype(jnp.bfloat16)
        ac[0] = jnp.zeros((Sq, D), jnp.bfloat16)
        st[0, :, 0:H] = jnp.full((Sq, H), -1e30, jnp.float32)
        st[0, :, H:2 * H] = jnp.zeros((Sq, H), jnp.float32)

        for s in range(N_DEV):
            for j in range(NQB):
                qs = pl.ds(j * BLK, BLK)
                gs = pl.ds(j * G, G)
                for h in range(H):
                    hs = slice(h * Dh, (h + 1) * Dh)
                    scores = lax.dot_general(
                        qc[s, qs, hs], kbf[gs, hs],
                        (((1,), (1,)), ((), ())),
                        preferred_element_type=jnp.float32)
                    m_old = st[s, qs, h:h + 1]
                    l_old = st[s, qs, H + h:H + h + 1]
                    m_new = jnp.maximum(
                        m_old, jnp.max(scores, axis=1, keepdims=True))
                    alpha = jnp.exp(m_old - m_new)
                    p = jnp.exp(scores - m_new)
                    l_new = l_old * alpha + jnp.sum(p, axis=1, keepdims=True)
                    pv = lax.dot_general(
                        p.astype(jnp.bfloat16), vbf[gs, hs],
                        (((1,), (0,)), ((), ())),
                        preferred_element_type=jnp.float32)
                    ac[s, qs, hs] = (ac[s, qs, hs].astype(jnp.float32) * alpha
                                     + pv).astype(jnp.bfloat16)
                    st[s, qs, h:h + 1] = m_new
                    st[s, qs, H + h:H + h + 1] = l_new

            rq = pltpu.make_async_remote_copy(
                src_ref=qc.at[s], dst_ref=qc.at[s + 1],
                send_sem=q_send.at[s], recv_sem=q_recv.at[s],
                device_id=(right,), device_id_type=pl.DeviceIdType.MESH)
            ra = pltpu.make_async_remote_copy(
                src_ref=ac.at[s], dst_ref=ac.at[s + 1],
                send_sem=a_send.at[s], recv_sem=a_recv.at[s],
                device_id=(right,), device_id_type=pl.DeviceIdType.MESH)
            rs = pltpu.make_async_remote_copy(
                src_ref=st.at[s], dst_ref=st.at[s + 1],
                send_sem=s_send.at[s], recv_sem=s_recv.at[s],
                device_id=(right,), device_id_type=pl.DeviceIdType.MESH)
            rq.start()
            ra.start()
            rs.start()
            rq.wait()
            ra.wait()
            rs.wait()

        for h in range(H):
            hs = slice(h * Dh, (h + 1) * Dh)
            ctx = (ac[N_DEV, :, hs].astype(jnp.float32)
                   / st[N_DEV, :, H + h:H + h + 1])
            ac[N_DEV, :, hs] = ctx.astype(jnp.bfloat16)
        out_ref[...] = jnp.dot(ac[N_DEV], wo_ref[...],
                               preferred_element_type=jnp.float32)

    out = pl.pallas_call(
        body,
        out_shape=jax.ShapeDtypeStruct((Sq, D), jnp.float32),
        in_specs=[
            pl.BlockSpec(memory_space=pltpu.VMEM),
            pl.BlockSpec(memory_space=pltpu.VMEM),
            pl.BlockSpec(memory_space=pltpu.VMEM),
            pl.BlockSpec(memory_space=pltpu.VMEM),
            pl.BlockSpec(memory_space=pltpu.VMEM),
        ],
        out_specs=pl.BlockSpec(memory_space=pltpu.VMEM),
        scratch_shapes=[
            pltpu.VMEM((N_DEV + 1, Sq, D), jnp.bfloat16),
            pltpu.VMEM((N_DEV + 1, Sq, D), jnp.bfloat16),
            pltpu.VMEM((N_DEV + 1, Sq, 2 * H), jnp.float32),
            pltpu.SemaphoreType.DMA((N_DEV,)),
            pltpu.SemaphoreType.DMA((N_DEV,)),
            pltpu.SemaphoreType.DMA((N_DEV,)),
            pltpu.SemaphoreType.DMA((N_DEV,)),
            pltpu.SemaphoreType.DMA((N_DEV,)),
            pltpu.SemaphoreType.DMA((N_DEV,)),
        ],
    )(x2, wq2, kp, vp, wo2)
    return out.reshape(B, Sq, D)


# device time: 77958 ns/iter; 2.2892x vs baseline; 1.9732x over previous
import jax
import jax.numpy as jnp
from jax import lax
from jax.experimental import pallas as pl
from jax.experimental.pallas import tpu as pltpu

N_DEV = 4
SCALE = 0.08838834764831843
BLK = 64
NGRP = 4


def kernel(x, Wq, K_ext, V_ext, Wo):
    B, Sq, D = x.shape
    _, Skv, H, Dh = K_ext.shape
    assert H * Dh == D
    G = Skv // NGRP
    NQB = Sq // BLK

    x2 = x.reshape(Sq, D).astype(jnp.bfloat16)
    wq2 = Wq.astype(jnp.bfloat16)
    wo2 = Wo.astype(jnp.bfloat16)
    kp = (K_ext.reshape(Skv // BLK // NGRP, NGRP, BLK, D)
          .transpose(1, 0, 2, 3).reshape(Skv, D).astype(jnp.bfloat16))
    vp = (V_ext.reshape(Skv // BLK // NGRP, NGRP, BLK, D)
          .transpose(1, 0, 2, 3).reshape(Skv, D).astype(jnp.bfloat16))

    def body(x_ref, wq_ref, kbf, vbf, wo_ref, out_ref,
             qg, pout, pacc, lout, lacc,
             q_send, q_recv, a_send, a_recv, l_send, l_recv):
        my = lax.axis_index("i")

        q = jnp.dot(x_ref[...], wq_ref[...], preferred_element_type=jnp.float32)
        qg[0] = (q * SCALE).astype(jnp.bfloat16)
        q_rdma = {}
        for d in (1, 2, 3):
            tgt = lax.rem(my + d, N_DEV)
            r = pltpu.make_async_remote_copy(
                src_ref=qg.at[0], dst_ref=qg.at[N_DEV - d],
                send_sem=q_send.at[d], recv_sem=q_recv.at[N_DEV - d],
                device_id=(tgt,), device_id_type=pl.DeviceIdType.MESH)
            r.start()
            q_rdma[d] = r

        p_rdma = []
        for d in range(N_DEV):
            if d > 0:
                pltpu.make_async_remote_copy(
                    src_ref=qg.at[0], dst_ref=qg.at[d],
                    send_sem=q_send.at[d], recv_sem=q_recv.at[d],
                    device_id=(my,), device_id_type=pl.DeviceIdType.MESH,
                ).wait_recv()
            adst = pacc if d == 0 else pout
            ldst = lacc if d == 0 else lout
            for j in range(NQB):
                qs = pl.ds(j * BLK, BLK)
                gs = pl.ds(j * G, G)
                for h in range(H):
                    hs = slice(h * Dh, (h + 1) * Dh)
                    scores = lax.dot_general(
                        qg[d, qs, hs], kbf[gs, hs],
                        (((1,), (1,)), ((), ())),
                        preferred_element_type=jnp.float32)
                    p = jnp.exp(scores).astype(jnp.bfloat16)
                    lsum = jnp.sum(p, axis=1, keepdims=True,
                                   dtype=jnp.float32)
                    pv = lax.dot_general(
                        p, vbf[gs, hs],
                        (((1,), (0,)), ((), ())),
                        preferred_element_type=jnp.float32)
                    adst[d, qs, hs] = pv.astype(jnp.bfloat16)
                    ldst[d, qs, h:h + 1] = lsum
            if d > 0:
                tgt = lax.rem(my + d, N_DEV)
                ra = pltpu.make_async_remote_copy(
                    src_ref=pout.at[d], dst_ref=pacc.at[N_DEV - d],
                    send_sem=a_send.at[d], recv_sem=a_recv.at[N_DEV - d],
                    device_id=(tgt,), device_id_type=pl.DeviceIdType.MESH)
                rl = pltpu.make_async_remote_copy(
                    src_ref=lout.at[d], dst_ref=lacc.at[N_DEV - d],
                    send_sem=l_send.at[d], recv_sem=l_recv.at[N_DEV - d],
                    device_id=(tgt,), device_id_type=pl.DeviceIdType.MESH)
                ra.start()
                rl.start()
                p_rdma.append((ra, rl))

        for r in (1, 2, 3):
            pltpu.make_async_remote_copy(
                src_ref=pout.at[1], dst_ref=pacc.at[r],
                send_sem=a_send.at[r], recv_sem=a_recv.at[r],
                device_id=(my,), device_id_type=pl.DeviceIdType.MESH,
            ).wait_recv()
            pltpu.make_async_remote_copy(
                src_ref=lout.at[1], dst_ref=lacc.at[r],
                send_sem=l_send.at[r], recv_sem=l_recv.at[r],
                device_id=(my,), device_id_type=pl.DeviceIdType.MESH,
            ).wait_recv()

        lsum = (lacc[0].astype(jnp.float32) + lacc[1].astype(jnp.float32)
                + lacc[2].astype(jnp.float32) + lacc[3].astype(jnp.float32))
        acc = (pacc[0].astype(jnp.float32) + pacc[1].astype(jnp.float32)
               + pacc[2].astype(jnp.float32) + pacc[3].astype(jnp.float32))
        for h in range(H):
            hs = slice(h * Dh, (h + 1) * Dh)
            pacc[0, :, hs] = (acc[:, hs] / lsum[:, h:h + 1]).astype(jnp.bfloat16)
        out_ref[...] = jnp.dot(pacc[0], wo_ref[...],
                               preferred_element_type=jnp.float32)

        for d in (1, 2, 3):
            q_rdma[d].wait_send()
        for ra, rl in p_rdma:
            ra.wait_send()
            rl.wait_send()

    out = pl.pallas_call(
        body,
        out_shape=jax.ShapeDtypeStruct((Sq, D), jnp.float32),
        in_specs=[
            pl.BlockSpec(memory_space=pltpu.VMEM),
            pl.BlockSpec(memory_space=pltpu.VMEM),
            pl.BlockSpec(memory_space=pltpu.VMEM),
            pl.BlockSpec(memory_space=pltpu.VMEM),
            pl.BlockSpec(memory_space=pltpu.VMEM),
        ],
        out_specs=pl.BlockSpec(memory_space=pltpu.VMEM),
        scratch_shapes=[
            pltpu.VMEM((N_DEV, Sq, D), jnp.bfloat16),
            pltpu.VMEM((N_DEV, Sq, D), jnp.bfloat16),
            pltpu.VMEM((N_DEV, Sq, D), jnp.bfloat16),
            pltpu.VMEM((N_DEV, Sq, H), jnp.float32),
            pltpu.VMEM((N_DEV, Sq, H), jnp.float32),
            pltpu.SemaphoreType.DMA((N_DEV,)),
            pltpu.SemaphoreType.DMA((N_DEV,)),
            pltpu.SemaphoreType.DMA((N_DEV,)),
            pltpu.SemaphoreType.DMA((N_DEV,)),
            pltpu.SemaphoreType.DMA((N_DEV,)),
            pltpu.SemaphoreType.DMA((N_DEV,)),
        ],
    )(x2, wq2, kp, vp, wo2)
    return out.reshape(B, Sq, D)


# device time: 64419 ns/iter; 2.7703x vs baseline; 1.2102x over previous
import jax
import jax.numpy as jnp
from jax import lax
from jax.experimental import pallas as pl
from jax.experimental.pallas import tpu as pltpu

N_DEV = 4
SCALE = 0.08838834764831843
BLK = 64
NGRP = 4


def kernel(x, Wq, K_ext, V_ext, Wo):
    B, Sq, D = x.shape
    _, Skv, H, Dh = K_ext.shape
    assert H * Dh == D
    G = Skv // NGRP
    NQB = Sq // BLK
    NPOS = Skv // BLK // NGRP

    x2 = x.reshape(Sq, D).astype(jnp.bfloat16)
    wq2 = Wq.astype(jnp.bfloat16)
    wo2 = Wo.astype(jnp.bfloat16)
    kp = (K_ext.reshape(NPOS, NGRP, BLK, H, Dh)
          .transpose(1, 3, 0, 2, 4).reshape(NGRP, H, G, Dh)
          .astype(jnp.bfloat16))
    vp = (V_ext.reshape(NPOS, NGRP, BLK, H, Dh)
          .transpose(1, 3, 0, 2, 4).reshape(NGRP, H, G, Dh)
          .astype(jnp.bfloat16))

    def body(x_ref, wq_ref, kbf, vbf, wo_ref, out_ref,
             qg, pout, pacc, lout, lacc, ctxb,
             q_send, q_recv, a_send, a_recv, l_send, l_recv):
        my = lax.axis_index("i")

        for h in range(H):
            hs = slice(h * Dh, (h + 1) * Dh)
            qh = jnp.dot(x_ref[...], wq_ref[:, hs],
                         preferred_element_type=jnp.float32)
            qg[0, h] = (qh * SCALE).astype(jnp.bfloat16)
        q_rdma = {}
        for d in (1, 2, 3):
            tgt = lax.rem(my + d, N_DEV)
            r = pltpu.make_async_remote_copy(
                src_ref=qg.at[0], dst_ref=qg.at[N_DEV - d],
                send_sem=q_send.at[d], recv_sem=q_recv.at[N_DEV - d],
                device_id=(tgt,), device_id_type=pl.DeviceIdType.MESH)
            r.start()
            q_rdma[d] = r

        for j in range(NQB):
            qs = pl.ds(j * BLK, BLK)
            for h in range(H):
                scores = lax.dot_general(
                    qg[0, h, qs, :], kbf[j, h],
                    (((1,), (1,)), ((), ())),
                    preferred_element_type=jnp.float32)
                p = jnp.exp(scores).astype(jnp.bfloat16)
                lsum = jnp.sum(p, axis=1, keepdims=True, dtype=jnp.float32)
                pv = lax.dot_general(
                    p, vbf[j, h],
                    (((1,), (0,)), ((), ())),
                    preferred_element_type=jnp.float32)
                pacc[0, h, qs, :] = pv.astype(jnp.bfloat16)
                lacc[0, qs, h:h + 1] = lsum

        for d in (1, 2, 3):
            pltpu.make_async_remote_copy(
                src_ref=qg.at[0], dst_ref=qg.at[d],
                send_sem=q_send.at[d], recv_sem=q_recv.at[d],
                device_id=(my,), device_id_type=pl.DeviceIdType.MESH,
            ).wait_recv()
        for j in range(NQB):
            qs = pl.ds(j * BLK, BLK)
            for h in range(H):
                qstack = jnp.concatenate(
                    [qg[1, h, qs, :], qg[2, h, qs, :], qg[3, h, qs, :]],
                    axis=0)
                scores = lax.dot_general(
                    qstack, kbf[j, h],
                    (((1,), (1,)), ((), ())),
                    preferred_element_type=jnp.float32)
                p = jnp.exp(scores).astype(jnp.bfloat16)
                lsum = jnp.sum(p, axis=1, keepdims=True, dtype=jnp.float32)
                pv = lax.dot_general(
                    p, vbf[j, h],
                    (((1,), (0,)), ((), ())),
                    preferred_element_type=jnp.float32)
                pvb = pv.astype(jnp.bfloat16)
                for d in (1, 2, 3):
                    pout[d, h, qs, :] = pvb[(d - 1) * BLK:d * BLK, :]
                    lout[d, qs, h:h + 1] = lsum[(d - 1) * BLK:d * BLK, :]

        p_rdma = []
        for d in (1, 2, 3):
            tgt = lax.rem(my + d, N_DEV)
            ra = pltpu.make_async_remote_copy(
                src_ref=pout.at[d], dst_ref=pacc.at[N_DEV - d],
                send_sem=a_send.at[d], recv_sem=a_recv.at[N_DEV - d],
                device_id=(tgt,), device_id_type=pl.DeviceIdType.MESH)
            rl = pltpu.make_async_remote_copy(
                src_ref=lout.at[d], dst_ref=lacc.at[N_DEV - d],
                send_sem=l_send.at[d], recv_sem=l_recv.at[N_DEV - d],
                device_id=(tgt,), device_id_type=pl.DeviceIdType.MESH)
            ra.start()
            rl.start()
            p_rdma.append((ra, rl))

        for r in (1, 2, 3):
            pltpu.make_async_remote_copy(
                src_ref=pout.at[1], dst_ref=pacc.at[r],
                send_sem=a_send.at[r], recv_sem=a_recv.at[r],
                device_id=(my,), device_id_type=pl.DeviceIdType.MESH,
            ).wait_recv()
            pltpu.make_async_remote_copy(
                src_ref=lout.at[1], dst_ref=lacc.at[r],
                send_sem=l_send.at[r], recv_sem=l_recv.at[r],
                device_id=(my,), device_id_type=pl.DeviceIdType.MESH,
            ).wait_recv()

        lsum = (lacc[0].astype(jnp.float32) + lacc[1].astype(jnp.float32)
                + lacc[2].astype(jnp.float32) + lacc[3].astype(jnp.float32))
        for h in range(H):
            hs = slice(h * Dh, (h + 1) * Dh)
            acc_h = (pacc[0, h].astype(jnp.float32)
                     + pacc[1, h].astype(jnp.float32)
                     + pacc[2, h].astype(jnp.float32)
                     + pacc[3, h].astype(jnp.float32))
            ctxb[:, hs] = (acc_h / lsum[:, h:h + 1]).astype(jnp.bfloat16)
        out_ref[...] = jnp.dot(ctxb[...], wo_ref[...],
                               preferred_element_type=jnp.float32)

        for d in (1, 2, 3):
            q_rdma[d].wait_send()
        for ra, rl in p_rdma:
            ra.wait_send()
            rl.wait_send()

    out = pl.pallas_call(
        body,
        out_shape=jax.ShapeDtypeStruct((Sq, D), jnp.float32),
        in_specs=[
            pl.BlockSpec(memory_space=pltpu.VMEM),
            pl.BlockSpec(memory_space=pltpu.VMEM),
            pl.BlockSpec(memory_space=pltpu.VMEM),
            pl.BlockSpec(memory_space=pltpu.VMEM),
            pl.BlockSpec(memory_space=pltpu.VMEM),
        ],
        out_specs=pl.BlockSpec(memory_space=pltpu.VMEM),
        scratch_shapes=[
            pltpu.VMEM((N_DEV, H, Sq, Dh), jnp.bfloat16),
            pltpu.VMEM((N_DEV, H, Sq, Dh), jnp.bfloat16),
            pltpu.VMEM((N_DEV, H, Sq, Dh), jnp.bfloat16),
            pltpu.VMEM((N_DEV, Sq, H), jnp.float32),
            pltpu.VMEM((N_DEV, Sq, H), jnp.float32),
            pltpu.VMEM((Sq, D), jnp.bfloat16),
            pltpu.SemaphoreType.DMA((N_DEV,)),
            pltpu.SemaphoreType.DMA((N_DEV,)),
            pltpu.SemaphoreType.DMA((N_DEV,)),
            pltpu.SemaphoreType.DMA((N_DEV,)),
            pltpu.SemaphoreType.DMA((N_DEV,)),
            pltpu.SemaphoreType.DMA((N_DEV,)),
        ],
    )(x2, wq2, kp, vp, wo2)
    return out.reshape(B, Sq, D)
